# Initial kernel scaffold; baseline (speedup 1.0000x reference)
import jax
import jax.numpy as jnp
from jax.experimental import pallas as pl


def kernel(atoms, neighbors, neighbor_mask):
    raise NotImplementedError("write your pallas kernel here")


# speedup vs baseline: 22.1123x; 22.1123x over previous
import functools

import jax
import jax.numpy as jnp
from jax import lax
from jax.experimental import pallas as pl
from jax.experimental.pallas import tpu as pltpu
from jax.experimental.pallas import tpu_sc as plsc

EPS = 1e-08

NC = 2
NS = 16
NW = NC * NS
L = 16


def _make_sc_kernel(B, A, N, CH):
    CPW = (B * A) // NW
    WPB = NW // B
    APW = A // WPB
    NCHUNK = APW // CH
    assert CPW == APW and NCHUNK * CH == APW and N % L == 0

    mesh = plsc.VectorSubcoreMesh(
        core_axis_name="c", subcore_axis_name="s",
        num_cores=NC, num_subcores=NS)

    @functools.partial(
        pl.kernel,
        out_type=(
            jax.ShapeDtypeStruct((B * A * N,), jnp.float32),
            jax.ShapeDtypeStruct((B * A * N * 3,), jnp.float32),
        ),
        mesh=mesh,
        compiler_params=pltpu.CompilerParams(needs_layout_passes=False),
        scratch_types=[
            pltpu.VMEM((A,), jnp.float32),
            pltpu.VMEM((A,), jnp.float32),
            pltpu.VMEM((A,), jnp.float32),
            pltpu.VMEM((CH * N,), jnp.int32),
            pltpu.VMEM((CH * N,), jnp.int32),
            pltpu.VMEM((CH * N,), jnp.float32),
            pltpu.VMEM((CH * N * 3,), jnp.float32),
        ],
    )
    def sc_kernel(atp_hbm, nbr_hbm, msk_hbm, dist_hbm, dvec_hbm,
                  tx, ty, tz, nb_v, mk_v, di_v, dv_v):
        cid = lax.axis_index("c")
        sid = lax.axis_index("s")
        wid = sid * NC + cid
        b = wid // WPB
        a0 = (wid % WPB) * APW

        tb = b * 3 * A
        pltpu.sync_copy(atp_hbm.at[pl.ds(tb, A)], tx)
        pltpu.sync_copy(atp_hbm.at[pl.ds(tb + A, A)], ty)
        pltpu.sync_copy(atp_hbm.at[pl.ds(tb + 2 * A, A)], tz)

        iota = lax.iota(jnp.int32, L)
        i3 = iota * 3

        def chunk_body(ck, carry):
            c0 = b * A + a0 + ck * CH
            off = c0 * N
            pltpu.sync_copy(nbr_hbm.at[pl.ds(off, CH * N)], nb_v)
            pltpu.sync_copy(msk_hbm.at[pl.ds(off, CH * N)], mk_v)

            def center_body(lc, carry2):
                a = a0 + ck * CH + lc
                av = jnp.full((L,), a, dtype=jnp.int32)
                cx = plsc.load_gather(tx, [av])
                cy = plsc.load_gather(ty, [av])
                cz = plsc.load_gather(tz, [av])
                for s in range(N // L):
                    o = lc * N + s * L
                    idxv = nb_v[pl.ds(o, L)]
                    mv = mk_v[pl.ds(o, L)]
                    gx = plsc.load_gather(tx, [idxv])
                    gy = plsc.load_gather(ty, [idxv])
                    gz = plsc.load_gather(tz, [idxv])
                    dx = gx - cx
                    dy = gy - cy
                    dz = gz - cz
                    ss = dx * dx + dy * dy + dz * dz
                    seed = jnp.int32(0x5F3759DF) - lax.shift_right_logical(
                        plsc.bitcast(ss, jnp.int32), 1)
                    y = plsc.bitcast(seed, jnp.float32)
                    h = ss * 0.5
                    y = y * (1.5 - h * y * y)
                    y = y * (1.5 - h * y * y)
                    d = ss * y
                    d = jnp.where(ss > 0.0, d, 0.0)
                    m = mv != 0
                    di_v[pl.ds(o, L)] = jnp.where(m, d, 0.0)
                    r = 1.0 / jnp.where(m, d + EPS, 1.0)
                    sidx = i3 + (o * 3)
                    plsc.store_scatter(dv_v, [sidx], dx * r)
                    plsc.store_scatter(dv_v, [sidx + 1], dy * r)
                    plsc.store_scatter(dv_v, [sidx + 2], dz * r)
                return carry2

            lax.fori_loop(0, CH, center_body, 0)
            pltpu.sync_copy(di_v, dist_hbm.at[pl.ds(off, CH * N)])
            pltpu.sync_copy(dv_v, dvec_hbm.at[pl.ds(off * 3, CH * N * 3)])
            return carry

        lax.fori_loop(0, NCHUNK, chunk_body, 0)

    return sc_kernel


def kernel(atoms, neighbors, neighbor_mask):
    B, A, _ = atoms.shape
    N = neighbors.shape[-1]
    atoms_p = jnp.swapaxes(atoms, 1, 2).reshape(B * 3 * A)
    nbr_flat = neighbors.reshape(B * A * N)
    msk_i32 = neighbor_mask.astype(jnp.int32).reshape(B * A * N)

    sc_kernel = _make_sc_kernel(B, A, N, CH=128)
    dist_flat, dvec_flat = sc_kernel(atoms_p, nbr_flat, msk_i32)

    distances = dist_flat.reshape(B, A, N)
    distance_vector = dvec_flat.reshape(B, A, N, 3)
    return (distances, distance_vector, neighbors, neighbor_mask)
